# baseline (speedup 1.0000x reference)
import functools

import jax
import jax.numpy as jnp
from jax import lax
from jax.experimental import pallas as pl
from jax.experimental.pallas import tpu as pltpu
from jax.experimental.pallas import tpu_sc as plsc

EMB = 32
NT = 4
SUB = 128
CHUNK = 256
NSUB = CHUNK // SUB
LANES = 16


def _make_lookup(n_rows):
    info = plsc.get_sparse_core_info()
    nw = info.num_cores * info.num_subcores
    n_per_w = n_rows // nw
    n_chunks = n_per_w // CHUNK
    assert n_per_w * nw == n_rows and n_chunks * CHUNK == n_per_w
    assert n_chunks % 2 == 0

    mesh = plsc.VectorSubcoreMesh(core_axis_name="c", subcore_axis_name="s")

    @functools.partial(
        pl.kernel,
        out_type=jax.ShapeDtypeStruct((n_rows, EMB), jnp.float32),
        mesh=mesh,
        scratch_types=[
            pltpu.VMEM((2, CHUNK * NT), jnp.int32),
            pltpu.VMEM((2, NT, NSUB, SUB), jnp.int32),
            pltpu.VMEM((2, NT, CHUNK, EMB), jnp.float32),
            pltpu.VMEM((2, CHUNK, EMB), jnp.float32),
            pltpu.SemaphoreType.DMA,
            pltpu.SemaphoreType.DMA,
            pltpu.SemaphoreType.DMA,
            pltpu.SemaphoreType.DMA,
            pltpu.SemaphoreType.DMA,
            pltpu.SemaphoreType.DMA,
        ],
        compiler_params=pltpu.CompilerParams(
            use_tc_tiling_on_sc=False, needs_layout_passes=False),
    )
    def lookup(idx_hbm, w0, w1, w2, w3, out_hbm,
               raw_v, idx_v, rows_v, out_v, gs0, gs1, is0, is1, os0, os1):
        tables = (w0, w1, w2, w3)
        gsem = (gs0, gs1)
        isem = (is0, is1)
        osem = (os0, os1)
        wid = lax.axis_index("s") * info.num_cores + lax.axis_index("c")
        base = wid * n_per_w
        lane4 = lax.iota(jnp.int32, LANES) * NT

        def raw_copy(k, sp):
            off = (base + k * CHUNK) * NT
            return pltpu.make_async_copy(
                idx_hbm.at[pl.ds(off, CHUNK * NT)], raw_v.at[sp], isem[sp])

        def deinterleave(sp):
            for t in range(NT):
                for m in range(NSUB):
                    for g in range(SUB // LANES):
                        src = lane4 + ((m * SUB + g * LANES) * NT + t)
                        v = plsc.load_gather(raw_v.at[sp], [src])
                        idx_v[sp, t, m, pl.ds(g * LANES, LANES)] = v

        def gather_copies(k, sp):
            del k
            return [pltpu.make_async_copy(
                tables[t].at[idx_v.at[sp, t, m]],
                rows_v.at[sp, t, pl.ds(m * SUB, SUB)], gsem[sp])
                for t in range(NT) for m in range(NSUB)]

        def out_copy(k, sp):
            off = base + k * CHUNK
            return pltpu.make_async_copy(
                out_v.at[sp], out_hbm.at[pl.ds(off, CHUNK)], osem[sp])

        raw_copy(0, 0).start()
        raw_copy(0, 0).wait()
        deinterleave(0)
        for c in gather_copies(0, 0):
            c.start()
        raw_copy(1, 1).start()

        def pair_body(kk, carry):
            for s in (0, 1):
                k = 2 * kk + s
                sn = 1 - s
                for c in gather_copies(k, s):
                    c.wait()

                @pl.when(k + 2 < n_chunks)
                def _prefetch_idx():
                    raw_copy(k + 2, s).start()

                @pl.when(k + 1 < n_chunks)
                def _fire_next():
                    raw_copy(k + 1, sn).wait()
                    deinterleave(sn)
                    for c in gather_copies(k + 1, sn):
                        c.start()

                @pl.when(k >= 2)
                def _reclaim_out():
                    out_copy(k - 2, s).wait()

                def row_body(j, carry2):
                    for h in (0, EMB // 2):
                        d = pl.ds(h, EMB // 2)
                        s01 = rows_v[s, 0, j, d] + rows_v[s, 1, j, d]
                        s23 = rows_v[s, 2, j, d] + rows_v[s, 3, j, d]
                        out_v[s, j, d] = (s01 + s23) * jnp.float32(0.25)
                    return carry2

                lax.fori_loop(0, CHUNK, row_body, 0, unroll=8)
                out_copy(k, s).start()
            return carry

        lax.fori_loop(0, n_chunks // 2, pair_body, 0)
        out_copy(n_chunks - 2, 0).wait()
        out_copy(n_chunks - 1, 1).wait()

    return lookup


def kernel(poi_path, W0, W1, W2, W3):
    b, h, nt = poi_path.shape
    n = b * h
    idx_flat = poi_path.reshape(n * nt)
    out = _make_lookup(n)(idx_flat, W0, W1, W2, W3)
    return out.reshape(b, h, EMB)


# speedup vs baseline: 1.8979x; 1.8979x over previous
import functools

import jax
import jax.numpy as jnp
from jax import lax
from jax.experimental import pallas as pl
from jax.experimental.pallas import tpu as pltpu
from jax.experimental.pallas import tpu_sc as plsc

EMB = 32
NT = 4
HIST = 200
ROW = HIST * NT
STREAMS = [(0, 128), (128, 128), (256, 128), (384, 128),
           (512, 128), (640, 128), (768, 32)]
LANES = 16


def _make_lookup(n_batch, table_rows):
    info = plsc.get_sparse_core_info()
    nw = info.num_cores * info.num_subcores
    b_per_w = n_batch // nw
    assert b_per_w * nw == n_batch and b_per_w % 2 == 0

    mesh = plsc.VectorSubcoreMesh(core_axis_name="c", subcore_axis_name="s")

    @functools.partial(
        pl.kernel,
        out_type=jax.ShapeDtypeStruct((n_batch, HIST * EMB), jnp.float32),
        mesh=mesh,
        scratch_types=[
            pltpu.VMEM((2, ROW), jnp.int32),
            pltpu.VMEM((2, ROW, EMB), jnp.float32),
            pltpu.VMEM((2, HIST * EMB), jnp.float32),
            pltpu.SemaphoreType.DMA,
            pltpu.SemaphoreType.DMA,
            pltpu.SemaphoreType.DMA,
            pltpu.SemaphoreType.DMA,
            pltpu.SemaphoreType.DMA,
            pltpu.SemaphoreType.DMA,
        ],
        compiler_params=pltpu.CompilerParams(use_tc_tiling_on_sc=False),
    )
    def lookup(idx_hbm, wcat, out_hbm,
               idx_v, rows_v, out_v, gs0, gs1, is0, is1, os0, os1):
        gsem = (gs0, gs1)
        isem = (is0, is1)
        osem = (os0, os1)
        wid = lax.axis_index("s") * info.num_cores + lax.axis_index("c")
        base = wid * b_per_w
        toffs = (lax.iota(jnp.int32, LANES) % NT) * table_rows

        def idx_copy(k, sp):
            return pltpu.make_async_copy(
                idx_hbm.at[base + k], idx_v.at[sp], isem[sp])

        def add_offsets(sp):
            for g in range(ROW // LANES):
                d = pl.ds(g * LANES, LANES)
                idx_v[sp, d] = idx_v[sp, d] + toffs

        def gather_copies(k, sp):
            del k
            return [pltpu.make_async_copy(
                wcat.at[idx_v.at[sp, pl.ds(off, ln)]],
                rows_v.at[sp, pl.ds(off, ln)], gsem[sp])
                for off, ln in STREAMS]

        def out_copy(k, sp):
            return pltpu.make_async_copy(
                out_v.at[sp], out_hbm.at[base + k], osem[sp])

        idx_copy(0, 0).start()
        idx_copy(0, 0).wait()
        add_offsets(0)
        for c in gather_copies(0, 0):
            c.start()
        idx_copy(1, 1).start()

        def pair_body(kk, carry):
            for s in (0, 1):
                k = 2 * kk + s
                sn = 1 - s
                for c in gather_copies(k, s):
                    c.wait()

                @pl.when(k + 2 < b_per_w)
                def _prefetch_idx():
                    idx_copy(k + 2, s).start()

                @pl.when(k + 1 < b_per_w)
                def _fire_next():
                    idx_copy(k + 1, sn).wait()
                    add_offsets(sn)
                    for c in gather_copies(k + 1, sn):
                        c.start()

                @pl.when(k >= 2)
                def _reclaim_out():
                    out_copy(k - 2, s).wait()

                def pos_body(j, carry2):
                    r = j * NT
                    o = j * EMB
                    for h in (0, EMB // 2):
                        d = pl.ds(h, EMB // 2)
                        s01 = rows_v[s, r, d] + rows_v[s, r + 1, d]
                        s23 = rows_v[s, r + 2, d] + rows_v[s, r + 3, d]
                        out_v[s, pl.ds(o + h, EMB // 2)] = (
                            (s01 + s23) * jnp.float32(0.25))
                    return carry2

                lax.fori_loop(0, HIST, pos_body, 0, unroll=8)
                out_copy(k, s).start()
            return carry

        lax.fori_loop(0, b_per_w // 2, pair_body, 0)
        out_copy(b_per_w - 2, 0).wait()
        out_copy(b_per_w - 1, 1).wait()

    return lookup


def kernel(poi_path, W0, W1, W2, W3):
    b, h, nt = poi_path.shape
    idx2d = poi_path.reshape(b, h * nt)
    wcat = jnp.concatenate([W0, W1, W2, W3], axis=0)
    out = _make_lookup(b, W0.shape[0])(idx2d, wcat)
    return out.reshape(b, h, EMB)
